# baseline (speedup 1.0000x reference)
import functools

import jax
import jax.numpy as jnp
from jax import lax
from jax.experimental import pallas as pl
from jax.experimental.pallas import tpu as pltpu
from jax.experimental.pallas import tpu_sc as plsc

N_NODES = 10000
N_EDGES = 320000
D_IN = 128
H1 = 64
H2 = 32

NC = 2
NS = 16
LANE = 16

NT = 10240
GROUP = 128
GROUPS_PER_TILE = 160
E_PAD = NS * GROUPS_PER_TILE * GROUP
E_ROWS = E_PAD // GROUP
G = 4
NB = GROUPS_PER_TILE // G
ROWS_PER_SUB = NT // NS


def _sc_agg_body(H, t1a_hbm, t1b_hbm, t2a_hbm, t2b_hbm,
                 src_hbm, dst_hbm, src2_hbm, dst2_hbm,
                 out_hbm, idx_src, idx_dst, rows, acc, t_sp, gsem, ssem):
    Hh = H // NC
    c = lax.axis_index("c")
    s = lax.axis_index("s")

    r0s = s * ROWS_PER_SUB

    def _stage(ta_hbm, tb_hbm, sync):
        @pl.when(c == 0)
        def _():
            cp = pltpu.async_copy(ta_hbm.at[pl.ds(r0s, ROWS_PER_SUB)],
                                  t_sp.at[pl.ds(r0s, ROWS_PER_SUB)], ssem)
            if sync:
                cp.wait()

        @pl.when(c == 1)
        def _():
            cp = pltpu.async_copy(tb_hbm.at[pl.ds(r0s, ROWS_PER_SUB)],
                                  t_sp.at[pl.ds(r0s, ROWS_PER_SUB)], ssem)
            if sync:
                cp.wait()

    _stage(t1a_hbm, t1b_hbm, False)

    def _zrow(i, carry):
        for k in range(Hh // LANE):
            rows[0, 0, i, pl.ds(k * LANE, LANE)] = jnp.zeros((LANE,),
                                                             jnp.float32)
        return carry
    lax.fori_loop(0, GROUP, _zrow, 0)

    def _zcp(i, carry):
        pltpu.sync_copy(rows.at[0, 0],
                        acc.at[pl.ds(r0s + i * GROUP, GROUP)])
        return carry
    lax.fori_loop(0, ROWS_PER_SUB // GROUP, _zcp, 0)
    pltpu.make_async_copy(t1a_hbm.at[pl.ds(0, ROWS_PER_SUB)],
                          t_sp.at[pl.ds(0, ROWS_PER_SUB)], ssem).wait()
    plsc.subcore_barrier()

    base = s * GROUPS_PER_TILE

    def _gathers(buf, b):
        for j in range(G):
            pltpu.async_copy(t_sp.at[idx_src.at[b * G + j]],
                             rows.at[buf, j], gsem)

    def _wait_gathers(buf):
        for j in range(G):
            pltpu.make_async_copy(t_sp.at[pl.ds(0, GROUP)],
                                  rows.at[buf, j], gsem).wait()

    def _scatters(buf, b):
        for j in range(G):
            pltpu.async_copy(rows.at[buf, j], acc.at[idx_dst.at[b * G + j]],
                             ssem, add=True)

    def _wait_scatters(buf):
        for j in range(G):
            pltpu.make_async_copy(rows.at[buf, j],
                                  acc.at[pl.ds(0, GROUP)], ssem).wait()

    def _process(sr_hbm, ds_hbm):
        pltpu.sync_copy(sr_hbm.at[pl.ds(base, GROUPS_PER_TILE)], idx_src)
        pltpu.sync_copy(ds_hbm.at[pl.ds(base, GROUPS_PER_TILE)], idx_dst)
        _gathers(0, 0)

        def _it(b, carry):
            cur = lax.rem(b, 2)
            nxt = 1 - cur
            _wait_gathers(cur)

            @pl.when(b + 1 < NB)
            def _():
                @pl.when(b >= 1)
                def _():
                    _wait_scatters(nxt)
                _gathers(nxt, b + 1)

            _scatters(cur, b)
            return carry
        lax.fori_loop(0, NB, _it, 0)
        _wait_scatters(0)
        _wait_scatters(1)

    _process(src_hbm, dst_hbm)
    plsc.subcore_barrier()
    _stage(t2a_hbm, t2b_hbm, True)
    plsc.subcore_barrier()
    _process(src2_hbm, dst2_hbm)
    plsc.subcore_barrier()

    def _wr(i, carry):
        r0 = r0s + i * GROUP
        pltpu.sync_copy(acc.at[pl.ds(r0, GROUP)], rows.at[0, 0])

        def _relu_row(r, cy):
            for k in range(Hh // LANE):
                sl = pl.ds(k * LANE, LANE)
                rows[0, 0, r, sl] = jnp.maximum(rows[0, 0, r, sl], 0.0)
            return cy
        lax.fori_loop(0, GROUP, _relu_row, 0)
        pltpu.sync_copy(rows.at[0, 0],
                        out_hbm.at[pl.ds(r0, GROUP), pl.ds(c * Hh, Hh)])
        return carry
    lax.fori_loop(0, ROWS_PER_SUB // GROUP, _wr, 0)


def _make_sc_agg(H):
    Hh = H // NC
    mesh = plsc.VectorSubcoreMesh(core_axis_name="c", subcore_axis_name="s",
                                  num_cores=NC, num_subcores=NS)
    return pl.kernel(
        functools.partial(_sc_agg_body, H),
        out_type=jax.ShapeDtypeStruct((NT, H), jnp.float32),
        mesh=mesh,
        scratch_types=[
            pltpu.VMEM((GROUPS_PER_TILE, GROUP), jnp.int32),
            pltpu.VMEM((GROUPS_PER_TILE, GROUP), jnp.int32),
            pltpu.VMEM((2, G, GROUP, Hh), jnp.float32),
            pltpu.VMEM_SHARED((NT, Hh), jnp.float32),
            pltpu.VMEM_SHARED((NT, Hh), jnp.float32),
            pltpu.SemaphoreType.DMA,
            pltpu.SemaphoreType.DMA,
        ],
        compiler_params=pltpu.CompilerParams(use_tc_tiling_on_sc=False),
        name=f"sc_rgcn_agg_h{H}",
    )


_sc_agg_h1 = _make_sc_agg(H1)
_sc_agg_h2 = _make_sc_agg(H2)


_BLK = 1024


def _mm_body(x_ref, w1_ref, w2_ref, o1_ref, o2_ref):
    x = x_ref[...]
    o1_ref[...] = jnp.dot(x, w1_ref[...], preferred_element_type=jnp.float32)
    o2_ref[...] = jnp.dot(x, w2_ref[...], preferred_element_type=jnp.float32)


def _tc_mm(x_pad, Wa, Wb):
    D, H = Wa.shape
    return pl.pallas_call(
        _mm_body,
        grid=(NT // _BLK,),
        in_specs=[
            pl.BlockSpec((_BLK, D), lambda i: (i, 0)),
            pl.BlockSpec((D, H), lambda i: (0, 0)),
            pl.BlockSpec((D, H), lambda i: (0, 0)),
        ],
        out_specs=[pl.BlockSpec((_BLK, H), lambda i: (i, 0))] * 2,
        out_shape=[jax.ShapeDtypeStruct((NT, H), jnp.float32)] * 2,
    )(x_pad, Wa, Wb)


def _prep_edges(edge_index):
    src = edge_index[0].astype(jnp.int32)
    dst = edge_index[1].astype(jnp.int32)
    pad = E_PAD - N_EDGES
    fill = N_NODES + (jnp.arange(pad, dtype=jnp.int32) % (NT - N_NODES))
    src = jnp.concatenate([src, fill]).reshape(E_ROWS, GROUP)
    dst = jnp.concatenate([dst, fill]).reshape(E_ROWS, GROUP)
    return src, dst


def _halves(t):
    h = t.shape[1] // NC
    return t[:, :h], t[:, h:]


def kernel(x, edge_index_1, edge_index_2, W1_1, W1_2, W2_1, W2_2):
    src1, dst1 = _prep_edges(edge_index_1)
    src2, dst2 = _prep_edges(edge_index_2)
    x_pad = jnp.pad(x, ((0, NT - N_NODES), (0, 0)))

    t1, t2 = _tc_mm(x_pad, W1_1, W1_2)
    t1a, t1b = _halves(t1)
    t2a, t2b = _halves(t2)
    h1 = _sc_agg_h1(t1a, t1b, t2a, t2b, src1, dst1, src2, dst2)
    u1, u2 = _tc_mm(h1, W2_1, W2_2)
    u1a, u1b = _halves(u1)
    u2a, u2b = _halves(u2)
    out = _sc_agg_h2(u1a, u1b, u2a, u2b, src1, dst1, src2, dst2)
    return out[:N_NODES]


# speedup vs baseline: 1.0391x; 1.0391x over previous
import functools

import jax
import jax.numpy as jnp
from jax import lax
from jax.experimental import pallas as pl
from jax.experimental.pallas import tpu as pltpu
from jax.experimental.pallas import tpu_sc as plsc

N_NODES = 10000
N_EDGES = 320000
D_IN = 128
H1 = 64
H2 = 32

NC = 2
NS = 16
LANE = 16

NT = 10240
GROUP = 128
GROUPS_PER_TILE = 160
E_PAD = NS * GROUPS_PER_TILE * GROUP
E_ROWS = E_PAD // GROUP
ROWS_PER_SUB = NT // NS


def _pipe_depth(Hh):
    return 5 if Hh >= 32 else 10


def _sc_agg_body(H, t1a_hbm, t1b_hbm, t2a_hbm, t2b_hbm,
                 src_hbm, dst_hbm, src2_hbm, dst2_hbm,
                 out_hbm, idx_src, idx_dst, rows, acc, t_sp, t2_sp,
                 gsem, ssem, stsem):
    Hh = H // NC
    G = _pipe_depth(Hh)
    NB = GROUPS_PER_TILE // G
    c = lax.axis_index("c")
    s = lax.axis_index("s")

    r0s = s * ROWS_PER_SUB

    def _stage(ta_hbm, tb_hbm, dst_sp):
        @pl.when(c == 0)
        def _():
            pltpu.async_copy(ta_hbm.at[pl.ds(r0s, ROWS_PER_SUB)],
                             dst_sp.at[pl.ds(r0s, ROWS_PER_SUB)], stsem)

        @pl.when(c == 1)
        def _():
            pltpu.async_copy(tb_hbm.at[pl.ds(r0s, ROWS_PER_SUB)],
                             dst_sp.at[pl.ds(r0s, ROWS_PER_SUB)], stsem)

    def _wait_stage(dst_sp):
        pltpu.make_async_copy(t1a_hbm.at[pl.ds(0, ROWS_PER_SUB)],
                              dst_sp.at[pl.ds(0, ROWS_PER_SUB)], stsem).wait()

    _stage(t1a_hbm, t1b_hbm, t_sp)

    def _zrow(i, carry):
        for k in range(Hh // LANE):
            rows[0, 0, i, pl.ds(k * LANE, LANE)] = jnp.zeros((LANE,),
                                                             jnp.float32)
        return carry
    lax.fori_loop(0, GROUP, _zrow, 0)

    def _zcp(i, carry):
        pltpu.sync_copy(rows.at[0, 0],
                        acc.at[pl.ds(r0s + i * GROUP, GROUP)])
        return carry
    lax.fori_loop(0, ROWS_PER_SUB // GROUP, _zcp, 0)
    _wait_stage(t_sp)
    plsc.subcore_barrier()
    if t2_sp.shape[0] == NT:
        _stage(t2a_hbm, t2b_hbm, t2_sp)

    base = s * GROUPS_PER_TILE

    def _gathers(tab, buf, b):
        for j in range(G):
            pltpu.async_copy(tab.at[idx_src.at[b * G + j]],
                             rows.at[buf, j], gsem)

    def _wait_gathers(buf):
        for j in range(G):
            pltpu.make_async_copy(t_sp.at[pl.ds(0, GROUP)],
                                  rows.at[buf, j], gsem).wait()

    def _scatters(buf, b):
        for j in range(G):
            pltpu.async_copy(rows.at[buf, j], acc.at[idx_dst.at[b * G + j]],
                             ssem, add=True)

    def _wait_scatters(buf):
        for j in range(G):
            pltpu.make_async_copy(rows.at[buf, j],
                                  acc.at[pl.ds(0, GROUP)], ssem).wait()

    def _process(sr_hbm, ds_hbm, tab):
        pltpu.sync_copy(sr_hbm.at[pl.ds(base, GROUPS_PER_TILE)], idx_src)
        pltpu.sync_copy(ds_hbm.at[pl.ds(base, GROUPS_PER_TILE)], idx_dst)
        _gathers(tab, 0, 0)

        def _it(b, carry):
            cur = lax.rem(b, 2)
            nxt = 1 - cur
            _wait_gathers(cur)

            @pl.when(b + 1 < NB)
            def _():
                @pl.when(b >= 1)
                def _():
                    _wait_scatters(nxt)
                _gathers(tab, nxt, b + 1)

            _scatters(cur, b)
            return carry
        lax.fori_loop(0, NB, _it, 0)
        _wait_scatters(0)
        _wait_scatters(1)

    double_tab = t2_sp.shape[0] == NT
    if double_tab:
        _process(src_hbm, dst_hbm, t_sp)
        _wait_stage(t2_sp)
        plsc.subcore_barrier()
        _process(src2_hbm, dst2_hbm, t2_sp)
    else:
        _process(src_hbm, dst_hbm, t_sp)
        plsc.subcore_barrier()
        _stage(t2a_hbm, t2b_hbm, t_sp)
        _wait_stage(t_sp)
        plsc.subcore_barrier()
        _process(src2_hbm, dst2_hbm, t_sp)
    plsc.subcore_barrier()

    def _wr(i, carry):
        r0 = r0s + i * GROUP
        pltpu.sync_copy(acc.at[pl.ds(r0, GROUP)], rows.at[0, 0])

        def _relu_row(r, cy):
            for k in range(Hh // LANE):
                sl = pl.ds(k * LANE, LANE)
                rows[0, 0, r, sl] = jnp.maximum(rows[0, 0, r, sl], 0.0)
            return cy
        lax.fori_loop(0, GROUP, _relu_row, 0)
        pltpu.sync_copy(rows.at[0, 0],
                        out_hbm.at[pl.ds(r0, GROUP), pl.ds(c * Hh, Hh)])
        return carry
    lax.fori_loop(0, ROWS_PER_SUB // GROUP, _wr, 0)


def _make_sc_agg(H):
    Hh = H // NC
    G = _pipe_depth(Hh)
    mesh = plsc.VectorSubcoreMesh(core_axis_name="c", subcore_axis_name="s",
                                  num_cores=NC, num_subcores=NS)
    return pl.kernel(
        functools.partial(_sc_agg_body, H),
        out_type=jax.ShapeDtypeStruct((NT, H), jnp.float32),
        mesh=mesh,
        scratch_types=[
            pltpu.VMEM((GROUPS_PER_TILE, GROUP), jnp.int32),
            pltpu.VMEM((GROUPS_PER_TILE, GROUP), jnp.int32),
            pltpu.VMEM((2, G, GROUP, Hh), jnp.float32),
            pltpu.VMEM_SHARED((NT, Hh), jnp.float32),
            pltpu.VMEM_SHARED((NT, Hh), jnp.float32),
            pltpu.VMEM_SHARED((NT if Hh < 32 else 8, Hh), jnp.float32),
            pltpu.SemaphoreType.DMA,
            pltpu.SemaphoreType.DMA,
            pltpu.SemaphoreType.DMA,
        ],
        compiler_params=pltpu.CompilerParams(use_tc_tiling_on_sc=False),
        name=f"sc_rgcn_agg_h{H}",
    )


_sc_agg_h1 = _make_sc_agg(H1)
_sc_agg_h2 = _make_sc_agg(H2)


_BLK = 1024


def _mm_body(x_ref, w1_ref, w2_ref, o1_ref, o2_ref):
    x = x_ref[...]
    o1_ref[...] = jnp.dot(x, w1_ref[...], preferred_element_type=jnp.float32)
    o2_ref[...] = jnp.dot(x, w2_ref[...], preferred_element_type=jnp.float32)


def _tc_mm(x_pad, Wa, Wb):
    D, H = Wa.shape
    return pl.pallas_call(
        _mm_body,
        grid=(NT // _BLK,),
        in_specs=[
            pl.BlockSpec((_BLK, D), lambda i: (i, 0)),
            pl.BlockSpec((D, H), lambda i: (0, 0)),
            pl.BlockSpec((D, H), lambda i: (0, 0)),
        ],
        out_specs=[pl.BlockSpec((_BLK, H), lambda i: (i, 0))] * 2,
        out_shape=[jax.ShapeDtypeStruct((NT, H), jnp.float32)] * 2,
    )(x_pad, Wa, Wb)


def _prep_edges(edge_index):
    src = edge_index[0].astype(jnp.int32)
    dst = edge_index[1].astype(jnp.int32)
    pad = E_PAD - N_EDGES
    fill = N_NODES + (jnp.arange(pad, dtype=jnp.int32) % (NT - N_NODES))
    src = jnp.concatenate([src, fill]).reshape(E_ROWS, GROUP)
    dst = jnp.concatenate([dst, fill]).reshape(E_ROWS, GROUP)
    return src, dst


def _halves(t):
    h = t.shape[1] // NC
    return t[:, :h], t[:, h:]


def kernel(x, edge_index_1, edge_index_2, W1_1, W1_2, W2_1, W2_2):
    src1, dst1 = _prep_edges(edge_index_1)
    src2, dst2 = _prep_edges(edge_index_2)
    x_pad = jnp.pad(x, ((0, NT - N_NODES), (0, 0)))

    t1, t2 = _tc_mm(x_pad, W1_1, W1_2)
    t1a, t1b = _halves(t1)
    t2a, t2b = _halves(t2)
    h1 = _sc_agg_h1(t1a, t1b, t2a, t2b, src1, dst1, src2, dst2)
    u1, u2 = _tc_mm(h1, W2_1, W2_2)
    u1a, u1b = _halves(u1)
    u2a, u2b = _halves(u2)
    out = _sc_agg_h2(u1a, u1b, u2a, u2b, src1, dst1, src2, dst2)
    return out[:N_NODES]


# speedup vs baseline: 1.0612x; 1.0212x over previous
import functools

import jax
import jax.numpy as jnp
import numpy as np
from jax import lax
from jax.experimental import pallas as pl
from jax.experimental.pallas import tpu as pltpu
from jax.experimental.pallas import tpu_sc as plsc

N_NODES = 10000
N_EDGES = 320000
D_IN = 128
H1 = 64
H2 = 32

NC = 2
NS = 16
LANE = 16

NT = 10240
GROUP = 128
GROUPS_PER_TILE = 160
E_PAD = NS * GROUPS_PER_TILE * GROUP
E_ROWS = E_PAD // GROUP
ROWS_PER_SUB = NT // NS


def _pipe_depth(Hh):
    return 5 if Hh >= 32 else 10


MAIN_ROWS = N_EDGES // GROUP
PAD_ROWS = E_ROWS - MAIN_ROWS
LAST_MAIN = MAIN_ROWS - (NS - 1) * GROUPS_PER_TILE


def _sc_agg_body(H, t1a_hbm, t1b_hbm, t2a_hbm, t2b_hbm,
                 src_hbm, dst_hbm, src2_hbm, dst2_hbm, pad_hbm,
                 out_hbm, idx_src, idx_dst, rows, acc, t_sp, t2_sp,
                 gsem, ssem, stsem):
    Hh = H // NC
    G = _pipe_depth(Hh)
    NB = GROUPS_PER_TILE // G
    c = lax.axis_index("c")
    s = lax.axis_index("s")

    r0s = s * ROWS_PER_SUB

    def _stage(ta_hbm, tb_hbm, dst_sp):
        @pl.when(c == 0)
        def _():
            pltpu.async_copy(ta_hbm.at[pl.ds(r0s, ROWS_PER_SUB)],
                             dst_sp.at[pl.ds(r0s, ROWS_PER_SUB)], stsem)

        @pl.when(c == 1)
        def _():
            pltpu.async_copy(tb_hbm.at[pl.ds(r0s, ROWS_PER_SUB)],
                             dst_sp.at[pl.ds(r0s, ROWS_PER_SUB)], stsem)

    def _wait_stage(dst_sp):
        pltpu.make_async_copy(t1a_hbm.at[pl.ds(0, ROWS_PER_SUB)],
                              dst_sp.at[pl.ds(0, ROWS_PER_SUB)], stsem).wait()

    _stage(t1a_hbm, t1b_hbm, t_sp)

    def _zrow(i, carry):
        for k in range(Hh // LANE):
            rows[0, 0, i, pl.ds(k * LANE, LANE)] = jnp.zeros((LANE,),
                                                             jnp.float32)
        return carry
    lax.fori_loop(0, GROUP, _zrow, 0)

    def _zcp(i, carry):
        pltpu.sync_copy(rows.at[0, 0],
                        acc.at[pl.ds(r0s + i * GROUP, GROUP)])
        return carry
    lax.fori_loop(0, ROWS_PER_SUB // GROUP, _zcp, 0)
    _wait_stage(t_sp)
    plsc.subcore_barrier()
    if t2_sp.shape[0] == NT:
        _stage(t2a_hbm, t2b_hbm, t2_sp)

    base = s * GROUPS_PER_TILE

    def _gathers(tab, buf, b):
        for j in range(G):
            pltpu.async_copy(tab.at[idx_src.at[b * G + j]],
                             rows.at[buf, j], gsem)

    def _wait_gathers(buf):
        for j in range(G):
            pltpu.make_async_copy(t_sp.at[pl.ds(0, GROUP)],
                                  rows.at[buf, j], gsem).wait()

    def _scatters(buf, b):
        for j in range(G):
            pltpu.async_copy(rows.at[buf, j], acc.at[idx_dst.at[b * G + j]],
                             ssem, add=True)

    def _wait_scatters(buf):
        for j in range(G):
            pltpu.make_async_copy(rows.at[buf, j],
                                  acc.at[pl.ds(0, GROUP)], ssem).wait()

    def _process(sr_hbm, ds_hbm, tab):
        @pl.when(s < NS - 1)
        def _():
            pltpu.sync_copy(sr_hbm.at[pl.ds(base, GROUPS_PER_TILE)], idx_src)
            pltpu.sync_copy(ds_hbm.at[pl.ds(base, GROUPS_PER_TILE)], idx_dst)

        @pl.when(s == NS - 1)
        def _():
            m0 = (NS - 1) * GROUPS_PER_TILE
            pltpu.sync_copy(sr_hbm.at[pl.ds(m0, LAST_MAIN)],
                            idx_src.at[pl.ds(0, LAST_MAIN)])
            pltpu.sync_copy(pad_hbm, idx_src.at[pl.ds(LAST_MAIN, PAD_ROWS)])
            pltpu.sync_copy(ds_hbm.at[pl.ds(m0, LAST_MAIN)],
                            idx_dst.at[pl.ds(0, LAST_MAIN)])
            pltpu.sync_copy(pad_hbm, idx_dst.at[pl.ds(LAST_MAIN, PAD_ROWS)])

        _gathers(tab, 0, 0)

        def _it(b, carry):
            cur = lax.rem(b, 2)
            nxt = 1 - cur
            _wait_gathers(cur)

            @pl.when(b + 1 < NB)
            def _():
                @pl.when(b >= 1)
                def _():
                    _wait_scatters(nxt)
                _gathers(tab, nxt, b + 1)

            _scatters(cur, b)
            return carry
        lax.fori_loop(0, NB, _it, 0)
        _wait_scatters(0)
        _wait_scatters(1)

    double_tab = t2_sp.shape[0] == NT
    if double_tab:
        _process(src_hbm, dst_hbm, t_sp)
        _wait_stage(t2_sp)
        plsc.subcore_barrier()
        _process(src2_hbm, dst2_hbm, t2_sp)
    else:
        _process(src_hbm, dst_hbm, t_sp)
        plsc.subcore_barrier()
        _stage(t2a_hbm, t2b_hbm, t_sp)
        _wait_stage(t_sp)
        plsc.subcore_barrier()
        _process(src2_hbm, dst2_hbm, t_sp)
    plsc.subcore_barrier()

    def _wr(i, carry):
        r0 = r0s + i * GROUP
        pltpu.sync_copy(acc.at[pl.ds(r0, GROUP)], rows.at[0, 0])

        def _relu_row(r, cy):
            for k in range(Hh // LANE):
                sl = pl.ds(k * LANE, LANE)
                rows[0, 0, r, sl] = jnp.maximum(rows[0, 0, r, sl], 0.0)
            return cy
        lax.fori_loop(0, GROUP, _relu_row, 0)
        pltpu.sync_copy(rows.at[0, 0],
                        out_hbm.at[pl.ds(r0, GROUP), pl.ds(c * Hh, Hh)])
        return carry
    lax.fori_loop(0, ROWS_PER_SUB // GROUP, _wr, 0)


def _make_sc_agg(H):
    Hh = H // NC
    G = _pipe_depth(Hh)
    mesh = plsc.VectorSubcoreMesh(core_axis_name="c", subcore_axis_name="s",
                                  num_cores=NC, num_subcores=NS)
    return pl.kernel(
        functools.partial(_sc_agg_body, H),
        out_type=jax.ShapeDtypeStruct((NT, H), jnp.float32),
        mesh=mesh,
        scratch_types=[
            pltpu.VMEM((GROUPS_PER_TILE, GROUP), jnp.int32),
            pltpu.VMEM((GROUPS_PER_TILE, GROUP), jnp.int32),
            pltpu.VMEM((2, G, GROUP, Hh), jnp.float32),
            pltpu.VMEM_SHARED((NT, Hh), jnp.float32),
            pltpu.VMEM_SHARED((NT, Hh), jnp.float32),
            pltpu.VMEM_SHARED((NT if Hh < 32 else 8, Hh), jnp.float32),
            pltpu.SemaphoreType.DMA,
            pltpu.SemaphoreType.DMA,
            pltpu.SemaphoreType.DMA,
        ],
        compiler_params=pltpu.CompilerParams(use_tc_tiling_on_sc=False),
        name=f"sc_rgcn_agg_h{H}",
    )


_sc_agg_h1 = _make_sc_agg(H1)
_sc_agg_h2 = _make_sc_agg(H2)


_BLK = 1024


def _mm_body(x_ref, w1_ref, w2_ref, o1a_ref, o1b_ref, o2a_ref, o2b_ref):
    x = x_ref[...]
    t1 = jnp.dot(x, w1_ref[...], preferred_element_type=jnp.float32)
    t2 = jnp.dot(x, w2_ref[...], preferred_element_type=jnp.float32)
    hh = t1.shape[1] // NC
    o1a_ref[...] = t1[:, :hh]
    o1b_ref[...] = t1[:, hh:]
    o2a_ref[...] = t2[:, :hh]
    o2b_ref[...] = t2[:, hh:]


def _tc_mm(x_pad, Wa, Wb):
    D, H = Wa.shape
    return pl.pallas_call(
        _mm_body,
        grid=(NT // _BLK,),
        in_specs=[
            pl.BlockSpec((_BLK, D), lambda i: (i, 0)),
            pl.BlockSpec((D, H), lambda i: (0, 0)),
            pl.BlockSpec((D, H), lambda i: (0, 0)),
        ],
        out_specs=[pl.BlockSpec((_BLK, H // NC), lambda i: (i, 0))] * 4,
        out_shape=[jax.ShapeDtypeStruct((NT, H // NC), jnp.float32)] * 4,
    )(x_pad, Wa, Wb)


_PAD_BLOCK = np.asarray(
    N_NODES + (np.arange(PAD_ROWS * GROUP) % (NT - N_NODES)),
    dtype=np.int32).reshape(PAD_ROWS, GROUP)


def _prep_edges(edge_index):
    src = edge_index[0].astype(jnp.int32).reshape(MAIN_ROWS, GROUP)
    dst = edge_index[1].astype(jnp.int32).reshape(MAIN_ROWS, GROUP)
    return src, dst


def kernel(x, edge_index_1, edge_index_2, W1_1, W1_2, W2_1, W2_2):
    src1, dst1 = _prep_edges(edge_index_1)
    src2, dst2 = _prep_edges(edge_index_2)
    pad_block = jnp.asarray(_PAD_BLOCK)
    x_pad = jnp.pad(x, ((0, NT - N_NODES), (0, 0)))

    t1a, t1b, t2a, t2b = _tc_mm(x_pad, W1_1, W1_2)
    h1 = _sc_agg_h1(t1a, t1b, t2a, t2b, src1, dst1, src2, dst2, pad_block)
    u1a, u1b, u2a, u2b = _tc_mm(h1, W2_1, W2_2)
    out = _sc_agg_h2(u1a, u1b, u2a, u2b, src1, dst1, src2, dst2, pad_block)
    return out[:N_NODES]


# speedup vs baseline: 1.0874x; 1.0247x over previous
import functools

import jax
import jax.numpy as jnp
import numpy as np
from jax import lax
from jax.experimental import pallas as pl
from jax.experimental.pallas import tpu as pltpu
from jax.experimental.pallas import tpu_sc as plsc

N_NODES = 10000
N_EDGES = 320000
D_IN = 128
H1 = 64
H2 = 32

NC = 2
NS = 16
LANE = 16

NT = 10240
GROUP = 128
GROUPS_PER_TILE = 160
E_PAD = NS * GROUPS_PER_TILE * GROUP
E_ROWS = E_PAD // GROUP
ROWS_PER_SUB = NT // NS


def _pipe_depth(Hh):
    return 5 if Hh >= 32 else 10


MAIN_ROWS = N_EDGES // GROUP
PAD_ROWS = E_ROWS - MAIN_ROWS
LAST_MAIN = MAIN_ROWS - (NS - 1) * GROUPS_PER_TILE


def _sc_agg_body(H, t1a_hbm, t1b_hbm, t2a_hbm, t2b_hbm,
                 src_hbm, dst_hbm, src2_hbm, dst2_hbm, pad_hbm,
                 out_hbm, idx_src, idx_dst, rows, acc, t_sp, t2_sp,
                 gsem, ssem, stsem):
    Hh = H // NC
    G = _pipe_depth(Hh)
    NB = GROUPS_PER_TILE // G
    c = lax.axis_index("c")
    s = lax.axis_index("s")

    r0s = s * ROWS_PER_SUB

    def _stage(ta_hbm, tb_hbm, dst_sp):
        @pl.when(c == 0)
        def _():
            pltpu.async_copy(ta_hbm.at[pl.ds(r0s, ROWS_PER_SUB)],
                             dst_sp.at[pl.ds(r0s, ROWS_PER_SUB)], stsem)

        @pl.when(c == 1)
        def _():
            pltpu.async_copy(tb_hbm.at[pl.ds(r0s, ROWS_PER_SUB)],
                             dst_sp.at[pl.ds(r0s, ROWS_PER_SUB)], stsem)

    def _wait_stage(dst_sp):
        pltpu.make_async_copy(t1a_hbm.at[pl.ds(0, ROWS_PER_SUB)],
                              dst_sp.at[pl.ds(0, ROWS_PER_SUB)], stsem).wait()

    _stage(t1a_hbm, t1b_hbm, t_sp)

    def _zrow(i, carry):
        for k in range(Hh // LANE):
            rows[0, 0, i, pl.ds(k * LANE, LANE)] = jnp.zeros((LANE,),
                                                             jnp.float32)
        return carry
    lax.fori_loop(0, GROUP, _zrow, 0)

    def _zcp(i, carry):
        pltpu.sync_copy(rows.at[0, 0],
                        acc.at[pl.ds(r0s + i * GROUP, GROUP)])
        return carry
    lax.fori_loop(0, ROWS_PER_SUB // GROUP, _zcp, 0)
    _wait_stage(t_sp)
    plsc.subcore_barrier()
    if t2_sp.shape[0] == NT:
        _stage(t2a_hbm, t2b_hbm, t2_sp)

    base = s * GROUPS_PER_TILE

    def _gathers(tab, buf, b):
        for j in range(G):
            pltpu.async_copy(tab.at[idx_src.at[b * G + j]],
                             rows.at[buf, j], gsem)

    def _wait_gathers(buf):
        for j in range(G):
            pltpu.make_async_copy(t_sp.at[pl.ds(0, GROUP)],
                                  rows.at[buf, j], gsem).wait()

    def _scatters(buf, b):
        for j in range(G):
            pltpu.async_copy(rows.at[buf, j], acc.at[idx_dst.at[b * G + j]],
                             ssem, add=True)

    def _wait_scatters(buf):
        for j in range(G):
            pltpu.make_async_copy(rows.at[buf, j],
                                  acc.at[pl.ds(0, GROUP)], ssem).wait()

    def _process(sr_hbm, ds_hbm, tab):
        @pl.when(s < NS - 1)
        def _():
            pltpu.sync_copy(sr_hbm.at[pl.ds(base, GROUPS_PER_TILE)], idx_src)
            pltpu.sync_copy(ds_hbm.at[pl.ds(base, GROUPS_PER_TILE)], idx_dst)

        @pl.when(s == NS - 1)
        def _():
            m0 = (NS - 1) * GROUPS_PER_TILE
            pltpu.sync_copy(sr_hbm.at[pl.ds(m0, LAST_MAIN)],
                            idx_src.at[pl.ds(0, LAST_MAIN)])
            pltpu.sync_copy(pad_hbm, idx_src.at[pl.ds(LAST_MAIN, PAD_ROWS)])
            pltpu.sync_copy(ds_hbm.at[pl.ds(m0, LAST_MAIN)],
                            idx_dst.at[pl.ds(0, LAST_MAIN)])
            pltpu.sync_copy(pad_hbm, idx_dst.at[pl.ds(LAST_MAIN, PAD_ROWS)])

        _gathers(tab, 0, 0)

        def _it(b, carry):
            cur = lax.rem(b, 2)
            nxt = 1 - cur
            _wait_gathers(cur)

            @pl.when(b + 1 < NB)
            def _():
                @pl.when(b >= 1)
                def _():
                    _wait_scatters(nxt)
                _gathers(tab, nxt, b + 1)

            _scatters(cur, b)
            return carry
        lax.fori_loop(0, NB, _it, 0)
        _wait_scatters(0)
        _wait_scatters(1)

    double_tab = t2_sp.shape[0] == NT
    if double_tab:
        _process(src_hbm, dst_hbm, t_sp)
        _wait_stage(t2_sp)
        plsc.subcore_barrier()
        _process(src2_hbm, dst2_hbm, t2_sp)
    else:
        _process(src_hbm, dst_hbm, t_sp)
        plsc.subcore_barrier()
        _stage(t2a_hbm, t2b_hbm, t_sp)
        _wait_stage(t_sp)
        plsc.subcore_barrier()
        _process(src2_hbm, dst2_hbm, t_sp)
    plsc.subcore_barrier()

    def _wr_chunk(r0, n):
        pltpu.sync_copy(acc.at[pl.ds(r0, n)], rows.at[0, 0, pl.ds(0, n)])

        def _relu_row(r, cy):
            for k in range(Hh // LANE):
                sl = pl.ds(k * LANE, LANE)
                rows[0, 0, r, sl] = jnp.maximum(rows[0, 0, r, sl], 0.0)
            return cy
        lax.fori_loop(0, n, _relu_row, 0)
        pltpu.sync_copy(rows.at[0, 0, pl.ds(0, n)],
                        out_hbm.at[pl.ds(r0, n), pl.ds(c * Hh, Hh)])

    @pl.when(s < NS - 1)
    def _():
        def _wr(i, carry):
            _wr_chunk(r0s + i * GROUP, GROUP)
            return carry
        lax.fori_loop(0, ROWS_PER_SUB // GROUP, _wr, 0)

    @pl.when(s == NS - 1)
    def _():
        last_full = (N_NODES - (NS - 1) * ROWS_PER_SUB) // GROUP
        tail = N_NODES - (NS - 1) * ROWS_PER_SUB - last_full * GROUP

        def _wr(i, carry):
            _wr_chunk((NS - 1) * ROWS_PER_SUB + i * GROUP, GROUP)
            return carry
        lax.fori_loop(0, last_full, _wr, 0)
        _wr_chunk((NS - 1) * ROWS_PER_SUB + last_full * GROUP, tail)


def _make_sc_agg(H):
    Hh = H // NC
    G = _pipe_depth(Hh)
    mesh = plsc.VectorSubcoreMesh(core_axis_name="c", subcore_axis_name="s",
                                  num_cores=NC, num_subcores=NS)
    return pl.kernel(
        functools.partial(_sc_agg_body, H),
        out_type=jax.ShapeDtypeStruct((N_NODES, H), jnp.float32),
        mesh=mesh,
        scratch_types=[
            pltpu.VMEM((GROUPS_PER_TILE, GROUP), jnp.int32),
            pltpu.VMEM((GROUPS_PER_TILE, GROUP), jnp.int32),
            pltpu.VMEM((2, G, GROUP, Hh), jnp.float32),
            pltpu.VMEM_SHARED((NT, Hh), jnp.float32),
            pltpu.VMEM_SHARED((NT, Hh), jnp.float32),
            pltpu.VMEM_SHARED((NT if Hh < 32 else 8, Hh), jnp.float32),
            pltpu.SemaphoreType.DMA,
            pltpu.SemaphoreType.DMA,
            pltpu.SemaphoreType.DMA,
        ],
        compiler_params=pltpu.CompilerParams(use_tc_tiling_on_sc=False),
        name=f"sc_rgcn_agg_h{H}",
    )


_sc_agg_h1 = _make_sc_agg(H1)
_sc_agg_h2 = _make_sc_agg(H2)


_BLK = 1024


def _mm_body(x_ref, w1_ref, w2_ref, o1a_ref, o1b_ref, o2a_ref, o2b_ref):
    x = x_ref[...]
    t1 = jnp.dot(x, w1_ref[...], preferred_element_type=jnp.float32)
    t2 = jnp.dot(x, w2_ref[...], preferred_element_type=jnp.float32)
    hh = t1.shape[1] // NC
    o1a_ref[...] = t1[:, :hh]
    o1b_ref[...] = t1[:, hh:]
    o2a_ref[...] = t2[:, :hh]
    o2b_ref[...] = t2[:, hh:]


def _tc_mm(x_pad, Wa, Wb):
    D, H = Wa.shape
    return pl.pallas_call(
        _mm_body,
        grid=(NT // _BLK,),
        in_specs=[
            pl.BlockSpec((_BLK, D), lambda i: (i, 0)),
            pl.BlockSpec((D, H), lambda i: (0, 0)),
            pl.BlockSpec((D, H), lambda i: (0, 0)),
        ],
        out_specs=[pl.BlockSpec((_BLK, H // NC), lambda i: (i, 0))] * 4,
        out_shape=[jax.ShapeDtypeStruct((NT, H // NC), jnp.float32)] * 4,
    )(x_pad, Wa, Wb)


_PAD_BLOCK = np.asarray(
    N_NODES + (np.arange(PAD_ROWS * GROUP) % (NT - N_NODES)),
    dtype=np.int32).reshape(PAD_ROWS, GROUP)


def _prep_edges(edge_index):
    src = edge_index[0].astype(jnp.int32).reshape(MAIN_ROWS, GROUP)
    dst = edge_index[1].astype(jnp.int32).reshape(MAIN_ROWS, GROUP)
    return src, dst


def kernel(x, edge_index_1, edge_index_2, W1_1, W1_2, W2_1, W2_2):
    src1, dst1 = _prep_edges(edge_index_1)
    src2, dst2 = _prep_edges(edge_index_2)
    pad_block = jnp.asarray(_PAD_BLOCK)

    t1a, t1b, t2a, t2b = _tc_mm(x, W1_1, W1_2)
    h1 = _sc_agg_h1(t1a, t1b, t2a, t2b, src1, dst1, src2, dst2, pad_block)
    u1a, u1b, u2a, u2b = _tc_mm(h1, W2_1, W2_2)
    return _sc_agg_h2(u1a, u1b, u2a, u2b, src1, dst1, src2, dst2, pad_block)
